# Initial kernel scaffold; baseline (speedup 1.0000x reference)
import jax
import jax.numpy as jnp
from jax.experimental import pallas as pl


def kernel(features, captions, emb, W_ih, W_hh, b_ih, b_hh, W_out, b_out, h0, c0):
    raise NotImplementedError("write your pallas kernel here")


# speedup vs baseline: 1.9246x; 1.9246x over previous
import jax
import jax.numpy as jnp
from jax.experimental import pallas as pl
from jax.experimental.pallas import tpu as pltpu
from jax.experimental.pallas import tpu_sc as plsc

_GATHER_WINDOW = 128
_ROW_SPLIT = 2


def _sc_gather_rows(table, idx_flat, width):
    n = idx_flat.shape[0]
    indices = idx_flat.reshape(1, n)
    mesh = plsc.VectorSubcoreMesh(core_axis_name="core", subcore_axis_name="subcore")

    @pl.kernel(
        out_type=jax.ShapeDtypeStruct((n, width), table.dtype),
        mesh=mesh,
    )
    def gather_kernel(x_hbm, i_hbm, o_hbm):
        def body(i_vmem, o_vmem):
            pltpu.sync_copy(x_hbm.at[i_vmem.at[0]], o_vmem)

        pltpu.emit_pipeline(
            body,
            grid=(n // _GATHER_WINDOW,),
            in_specs=[pl.BlockSpec((1, _GATHER_WINDOW), lambda i: (0, i))],
            out_specs=[pl.BlockSpec((_GATHER_WINDOW, width), lambda i: (i, 0))],
            core_axis_name=("core", "subcore"),
            dimension_semantics=(pltpu.PARALLEL,),
        )(i_hbm, o_hbm)

    return gather_kernel(table, indices)


def _lstm_decode(features, embeds_rest, WihT, WhhT, b, WoutT, b_out, h0, c0):
    Lm1, B, E = embeds_rest.shape
    L = Lm1 + 1
    H = WhhT.shape[0]
    V = WoutT.shape[1]

    def step_kernel(x_ref, feat_ref, wih_ref, whh_ref, b_ref, wout_ref,
                    bout_ref, h0_ref, c0_ref, out_ref, h_ref, c_ref):
        t = pl.program_id(0)

        @pl.when(t == 0)
        def _init():
            h_ref[...] = h0_ref[...]
            c_ref[...] = c0_ref[...]

        x = jnp.where(t == 0, feat_ref[...], x_ref[0])
        gates = (
            jnp.dot(x, wih_ref[...], preferred_element_type=jnp.float32)
            + jnp.dot(h_ref[...], whh_ref[...], preferred_element_type=jnp.float32)
            + b_ref[...]
        )
        i = jax.nn.sigmoid(gates[:, 0 * H:1 * H])
        f = jax.nn.sigmoid(gates[:, 1 * H:2 * H])
        g = jnp.tanh(gates[:, 2 * H:3 * H])
        o = jax.nn.sigmoid(gates[:, 3 * H:4 * H])
        c = f * c_ref[...] + i * g
        h = o * jnp.tanh(c)
        c_ref[...] = c
        h_ref[...] = h
        out_ref[0] = (
            jnp.dot(h, wout_ref[...], preferred_element_type=jnp.float32)
            + bout_ref[...]
        )

    return pl.pallas_call(
        step_kernel,
        grid=(L,),
        in_specs=[
            pl.BlockSpec((1, B, E), lambda t: (jnp.maximum(t - 1, 0), 0, 0)),
            pl.BlockSpec((B, E), lambda t: (0, 0)),
            pl.BlockSpec((E, 4 * H), lambda t: (0, 0)),
            pl.BlockSpec((H, 4 * H), lambda t: (0, 0)),
            pl.BlockSpec((1, 4 * H), lambda t: (0, 0)),
            pl.BlockSpec((H, V), lambda t: (0, 0)),
            pl.BlockSpec((1, V), lambda t: (0, 0)),
            pl.BlockSpec((B, H), lambda t: (0, 0)),
            pl.BlockSpec((B, H), lambda t: (0, 0)),
        ],
        out_specs=pl.BlockSpec((1, B, V), lambda t: (t, 0, 0)),
        out_shape=jax.ShapeDtypeStruct((L, B, V), jnp.float32),
        scratch_shapes=[
            pltpu.VMEM((B, H), jnp.float32),
            pltpu.VMEM((B, H), jnp.float32),
        ],
        compiler_params=pltpu.CompilerParams(
            dimension_semantics=("arbitrary",),
        ),
    )(embeds_rest, features, WihT, WhhT, b, WoutT, b_out, h0, c0)


def kernel(features, captions, emb, W_ih, W_hh, b_ih, b_hh, W_out, b_out, h0, c0):
    B, L = captions.shape
    E = emb.shape[1]

    idx = jnp.transpose(captions[:, :-1]).reshape(-1)
    s = _ROW_SPLIT
    idx_split = (idx[:, None] * s
                 + jnp.arange(s, dtype=idx.dtype)[None, :]).reshape(-1)
    table = emb.reshape(emb.shape[0] * s, E // s)
    gathered = _sc_gather_rows(table, idx_split, E // s)
    embeds_rest = gathered.reshape(L - 1, B, E)

    WihT = jnp.transpose(W_ih)
    WhhT = jnp.transpose(W_hh)
    b = (b_ih + b_hh).reshape(1, -1)
    WoutT = jnp.transpose(W_out)
    b_out2 = b_out.reshape(1, -1)

    logits_tm = _lstm_decode(features, embeds_rest, WihT, WhhT, b,
                             WoutT, b_out2, h0[0], c0[0])
    return jnp.transpose(logits_tm, (1, 0, 2))
